# baseline (speedup 1.0000x reference)
import functools

import jax
import jax.numpy as jnp
from jax import lax
from jax.experimental import pallas as pl
from jax.experimental.pallas import tpu as pltpu
from jax.experimental.pallas import tpu_sc as plsc

_BK = 256
_LANES = 16


def _dist_body(x_ref, w_ref, bv_ref, bi_ref, *, nb):
    q = x_ref[...]
    w = w_ref[...]
    w2 = jnp.sum(w * w, axis=1)
    dot = jnp.dot(
        w, q.T,
        precision=jax.lax.Precision.HIGHEST,
        preferred_element_type=jnp.float32,
    )
    d2 = w2[:, None] - 2.0 * dot
    mins, args = [], []
    for e in range(nb):
        sub = d2[e * _BK:(e + 1) * _BK, :]
        mins.append(jnp.min(sub, axis=0))
        args.append(jnp.argmin(sub, axis=0).astype(jnp.int32) + e * _BK)
    bv_ref[...] = jnp.stack(mins, axis=0)
    bi_ref[...] = jnp.stack(args, axis=0)


def _merge_kernel(bv_hbm, bi_hbm, o_hbm, v_vmem, i_vmem, o_vmem, sem0, sem1,
                  *, nb, chunk, n_workers, cols):
    wid = lax.axis_index("c") + 2 * lax.axis_index("s")

    @pl.when(wid < n_workers)
    def _():
        base = wid * chunk
        cp0 = pltpu.async_copy(bv_hbm.at[:, pl.ds(base, chunk)], v_vmem, sem0)
        cp1 = pltpu.async_copy(bi_hbm.at[:, pl.ds(base, chunk)], i_vmem, sem1)
        cp0.wait()
        cp1.wait()
        for g in range(0, chunk, _LANES):
            sl = pl.ds(g, _LANES)
            m = v_vmem[0, sl]
            mi = i_vmem[0, sl]
            for e in range(1, nb):
                v = v_vmem[e, sl]
                take = v < m
                m = jnp.where(take, v, m)
                mi = jnp.where(take, i_vmem[e, sl], mi)
            o_vmem[0, sl] = mi // cols
            o_vmem[1, sl] = mi % cols
        pltpu.sync_copy(o_vmem, o_hbm.at[:, pl.ds(base, chunk)])


def kernel(x, weights):
    rows, cols, d = weights.shape
    b = x.shape[0]
    k = rows * cols
    nb = k // _BK
    w = weights.reshape(k, d)
    tb = min(b, 1024)
    dist = pl.pallas_call(
        functools.partial(_dist_body, nb=nb),
        grid=(b // tb,),
        in_specs=[
            pl.BlockSpec((tb, d), lambda i: (i, 0)),
            pl.BlockSpec((k, d), lambda i: (0, 0)),
        ],
        out_specs=[
            pl.BlockSpec((nb, tb), lambda i: (0, i)),
            pl.BlockSpec((nb, tb), lambda i: (0, i)),
        ],
        out_shape=[
            jax.ShapeDtypeStruct((nb, b), jnp.float32),
            jax.ShapeDtypeStruct((nb, b), jnp.int32),
        ],
    )(x, w)
    bv, bi = dist

    chunk = 128
    n_workers = b // chunk
    mesh = plsc.VectorSubcoreMesh(core_axis_name="c", subcore_axis_name="s")
    merge = pl.kernel(
        functools.partial(_merge_kernel, nb=nb, chunk=chunk,
                          n_workers=n_workers, cols=cols),
        out_type=jax.ShapeDtypeStruct((2, b), jnp.int32),
        mesh=mesh,
        compiler_params=pltpu.CompilerParams(needs_layout_passes=False),
        scratch_types=[
            pltpu.VMEM((nb, chunk), jnp.float32),
            pltpu.VMEM((nb, chunk), jnp.int32),
            pltpu.VMEM((2, chunk), jnp.int32),
            pltpu.SemaphoreType.DMA,
            pltpu.SemaphoreType.DMA,
        ],
    )
    rc = merge(bv, bi)
    return rc.T


# speedup vs baseline: 1.0049x; 1.0049x over previous
import functools

import jax
import jax.numpy as jnp
from jax import lax
from jax.experimental import pallas as pl
from jax.experimental.pallas import tpu as pltpu
from jax.experimental.pallas import tpu_sc as plsc

_BK = 256
_LANES = 16


def _dist_body(x_ref, w_ref, bv_ref, bi_ref, *, nb):
    q = x_ref[...]
    w = w_ref[...]
    w2 = jnp.sum(w * w, axis=1)
    dot = jnp.dot(
        w, q.T,
        precision=jax.lax.Precision.HIGHEST,
        preferred_element_type=jnp.float32,
    )
    d2 = w2[:, None] - 2.0 * dot
    mins, args = [], []
    for e in range(nb):
        sub = d2[e * _BK:(e + 1) * _BK, :]
        mins.append(jnp.min(sub, axis=0))
        args.append(jnp.argmin(sub, axis=0).astype(jnp.int32) + e * _BK)
    bv_ref[...] = jnp.stack(mins, axis=0)
    bi_ref[...] = jnp.stack(args, axis=0)


def _merge_kernel(bv_hbm, bi_hbm, o_hbm, v_vmem, i_vmem, o_vmem, sem0, sem1,
                  *, nb, chunk, n_workers, cols):
    wid = lax.axis_index("c") + 2 * lax.axis_index("s")

    @pl.when(wid < n_workers)
    def _():
        base = wid * chunk
        cp0 = pltpu.async_copy(bv_hbm.at[:, pl.ds(base, chunk)], v_vmem, sem0)
        cp1 = pltpu.async_copy(bi_hbm.at[:, pl.ds(base, chunk)], i_vmem, sem1)
        cp0.wait()
        cp1.wait()
        for g in range(0, chunk, _LANES):
            sl = pl.ds(g, _LANES)
            m = v_vmem[0, sl]
            mi = i_vmem[0, sl]
            for e in range(1, nb):
                v = v_vmem[e, sl]
                take = v < m
                m = jnp.where(take, v, m)
                mi = jnp.where(take, i_vmem[e, sl], mi)
            o_vmem[0, sl] = mi // cols
            o_vmem[1, sl] = mi % cols
        pltpu.sync_copy(o_vmem, o_hbm.at[:, pl.ds(base, chunk)])


def kernel(x, weights):
    rows, cols, d = weights.shape
    b = x.shape[0]
    k = rows * cols
    nb = k // _BK
    w = weights.reshape(k, d)
    tb = min(b, 256)
    dist = pl.pallas_call(
        functools.partial(_dist_body, nb=nb),
        grid=(b // tb,),
        in_specs=[
            pl.BlockSpec((tb, d), lambda i: (i, 0)),
            pl.BlockSpec((k, d), lambda i: (0, 0)),
        ],
        out_specs=[
            pl.BlockSpec((nb, tb), lambda i: (0, i)),
            pl.BlockSpec((nb, tb), lambda i: (0, i)),
        ],
        out_shape=[
            jax.ShapeDtypeStruct((nb, b), jnp.float32),
            jax.ShapeDtypeStruct((nb, b), jnp.int32),
        ],
    )(x, w)
    bv, bi = dist

    chunk = 128
    n_workers = b // chunk
    mesh = plsc.VectorSubcoreMesh(core_axis_name="c", subcore_axis_name="s")
    merge = pl.kernel(
        functools.partial(_merge_kernel, nb=nb, chunk=chunk,
                          n_workers=n_workers, cols=cols),
        out_type=jax.ShapeDtypeStruct((2, b), jnp.int32),
        mesh=mesh,
        compiler_params=pltpu.CompilerParams(needs_layout_passes=False),
        scratch_types=[
            pltpu.VMEM((nb, chunk), jnp.float32),
            pltpu.VMEM((nb, chunk), jnp.int32),
            pltpu.VMEM((2, chunk), jnp.int32),
            pltpu.SemaphoreType.DMA,
            pltpu.SemaphoreType.DMA,
        ],
    )
    rc = merge(bv, bi)
    return rc.T
